# baseline (speedup 1.0000x reference)
import functools

import jax
import jax.numpy as jnp
from jax import lax
from jax.experimental import pallas as pl
from jax.experimental.pallas import tpu as pltpu
from jax.experimental.pallas import tpu_sc as plsc

B = 1024
CTX = 20
EMB = 16
VOCAB_FLAT = 100000 * EMB
NW = 32
BAGS_PER_W = B // NW
ROWS_PER_W = BAGS_PER_W * CTX
IDX_CHUNK = 128
N_CHUNKS = ROWS_PER_W // IDX_CHUNK


def _sc_avg_body(bags_hbm, table_hbm, out_hbm, idx_v, rows_v, avg_v, sem):
    wid = lax.axis_index("s") * 2 + lax.axis_index("c")
    pltpu.sync_copy(bags_hbm.at[wid], idx_v)
    copies = []
    for j in range(N_CHUNKS):
        copies.append(
            pltpu.async_copy(
                table_hbm.at[idx_v.at[j]],
                rows_v.at[pl.ds(j * IDX_CHUNK, IDX_CHUNK)],
                sem,
            )
        )
    for c in copies:
        c.wait()

    inv = jnp.full((EMB,), 1.0 / CTX, dtype=jnp.float32)

    def pool_one(i, _):
        acc = rows_v[i * CTX]
        for j in range(1, CTX):
            acc = acc + rows_v[i * CTX + j]
        avg_v[i] = acc * inv
        return _

    lax.fori_loop(0, BAGS_PER_W, pool_one, 0, unroll=4)
    pltpu.sync_copy(avg_v, out_hbm.at[pl.ds(wid * BAGS_PER_W, BAGS_PER_W)])


@jax.jit
def _sc_avg(bags_r, bag_emb):
    mesh = plsc.VectorSubcoreMesh(core_axis_name="c", subcore_axis_name="s")
    return pl.kernel(
        _sc_avg_body,
        out_type=jax.ShapeDtypeStruct((B, EMB), jnp.float32),
        mesh=mesh,
        scratch_types=[
            pltpu.VMEM((N_CHUNKS, IDX_CHUNK), jnp.int32),
            pltpu.VMEM((ROWS_PER_W, EMB), jnp.float32),
            pltpu.VMEM((BAGS_PER_W, EMB), jnp.float32),
            pltpu.SemaphoreType.DMA,
        ],
        compiler_params=pltpu.CompilerParams(use_tc_tiling_on_sc=False),
    )(bags_r, bag_emb)


TR = 12544
TPB = 1792
TNB = TR // TPB


def _relayout_body(*refs):
    ins, out_ref = refs[:-1], refs[-1]
    y = jnp.concatenate([r[...] for r in ins], axis=0)
    out_ref[...] = y.T


@jax.jit
def _tc_table_rows(table_t):
    specs = [
        pl.BlockSpec((EMB, TPB), (lambda i, a=a: (0, a * TNB + i)))
        for a in range(8)
    ]
    return pl.pallas_call(
        _relayout_body,
        grid=(TNB,),
        in_specs=specs,
        out_specs=pl.BlockSpec((TPB, 128), lambda i: (i, 0)),
        out_shape=jax.ShapeDtypeStruct((TR, 128), jnp.float32),
    )(*([table_t] * 8))


VB = 1024


def _matmul_body(tag_ref, avg_ref, out_ref):
    out_ref[...] = lax.dot_general(
        tag_ref[...],
        avg_ref[...],
        (((0,), (1,)), ((), ())),
        preferred_element_type=jnp.float32,
    )


@jax.jit
def _tc_scores_t(tag_t, avg):
    vocab = tag_t.shape[1]
    grid = (pl.cdiv(vocab, VB),)
    return pl.pallas_call(
        _matmul_body,
        grid=grid,
        in_specs=[
            pl.BlockSpec((EMB, VB), lambda i: (0, i)),
            pl.BlockSpec((B, EMB), lambda i: (0, 0)),
        ],
        out_specs=pl.BlockSpec((VB, B), lambda i: (i, 0)),
        out_shape=jax.ShapeDtypeStruct((vocab, B), jnp.float32),
    )(tag_t, avg)


def kernel(bags, bag_emb, tag_emb):
    fake = _tc_table_rows(bag_emb.T)
    table_rows = fake.reshape(TR * 8, EMB)
    bags_flat = bags.reshape(B * CTX)
    ridx = (bags_flat % TR) * 8 + bags_flat // TR
    ridx_r = ridx.reshape(NW, N_CHUNKS, IDX_CHUNK)
    avg = _sc_avg(ridx_r, table_rows)
    return _tc_scores_t(tag_emb.T, avg).T


# speedup vs baseline: 1.1066x; 1.1066x over previous
import functools

import jax
import jax.numpy as jnp
from jax import lax
from jax.experimental import pallas as pl
from jax.experimental.pallas import tpu as pltpu
from jax.experimental.pallas import tpu_sc as plsc

B = 1024
CTX = 20
EMB = 16
VOCAB_FLAT = 100000 * EMB
NW = 32
BAGS_PER_W = B // NW
ROWS_PER_W = BAGS_PER_W * CTX
IDX_CHUNK = 128
N_CHUNKS = ROWS_PER_W // IDX_CHUNK


def _sc_avg_body(bags_hbm, table_hbm, out_hbm, idx_v, rows_v, avg_v, sem):
    wid = lax.axis_index("s") * 2 + lax.axis_index("c")
    pltpu.sync_copy(bags_hbm.at[wid], idx_v)
    copies = []
    for j in range(N_CHUNKS):
        copies.append(
            pltpu.async_copy(
                table_hbm.at[idx_v.at[j]],
                rows_v.at[pl.ds(j * IDX_CHUNK, IDX_CHUNK)],
                sem,
            )
        )
    for c in copies:
        c.wait()

    inv = jnp.full((EMB,), 1.0 / CTX, dtype=jnp.float32)

    def pool_one(i, _):
        acc = rows_v[i * CTX]
        for j in range(1, CTX):
            acc = acc + rows_v[i * CTX + j]
        avg_v[i] = acc * inv
        return _

    lax.fori_loop(0, BAGS_PER_W, pool_one, 0, unroll=4)
    pltpu.sync_copy(avg_v, out_hbm.at[pl.ds(wid * BAGS_PER_W, BAGS_PER_W)])


@jax.jit
def _sc_avg(bags_r, bag_emb):
    mesh = plsc.VectorSubcoreMesh(core_axis_name="c", subcore_axis_name="s")
    return pl.kernel(
        _sc_avg_body,
        out_type=jax.ShapeDtypeStruct((B, EMB), jnp.float32),
        mesh=mesh,
        scratch_types=[
            pltpu.VMEM((N_CHUNKS, IDX_CHUNK), jnp.int32),
            pltpu.VMEM((ROWS_PER_W, EMB), jnp.float32),
            pltpu.VMEM((BAGS_PER_W, EMB), jnp.float32),
            pltpu.SemaphoreType.DMA,
        ],
        compiler_params=pltpu.CompilerParams(use_tc_tiling_on_sc=False),
    )(bags_r, bag_emb)


TR = 16384
TPB = 2048
TNB = TR // TPB


def _relayout_body(*refs):
    ins, out_ref = refs[:-1], refs[-1]
    y = jnp.concatenate([r[...] for r in ins], axis=0)
    out_ref[...] = y.T


@jax.jit
def _tc_table_rows(table_t):
    nblk = pl.cdiv(table_t.shape[1], TPB) - 1
    specs = [
        pl.BlockSpec(
            (EMB, TPB), (lambda i, a=a: (0, jnp.minimum(a * TNB + i, nblk)))
        )
        for a in range(8)
    ]
    return pl.pallas_call(
        _relayout_body,
        grid=(TNB,),
        in_specs=specs,
        out_specs=pl.BlockSpec((TPB, 128), lambda i: (i, 0)),
        out_shape=jax.ShapeDtypeStruct((TR, 128), jnp.float32),
    )(*([table_t] * 8))


VB = 2048


def _matmul_body(tag_ref, avg_ref, out_ref):
    out_ref[...] = lax.dot_general(
        tag_ref[...],
        avg_ref[...],
        (((0,), (1,)), ((), ())),
        preferred_element_type=jnp.float32,
    )


@jax.jit
def _tc_scores_t(tag_t, avg):
    vocab = tag_t.shape[1]
    grid = (pl.cdiv(vocab, VB),)
    return pl.pallas_call(
        _matmul_body,
        grid=grid,
        in_specs=[
            pl.BlockSpec((EMB, VB), lambda i: (0, i)),
            pl.BlockSpec((B, EMB), lambda i: (0, 0)),
        ],
        out_specs=pl.BlockSpec((VB, B), lambda i: (i, 0)),
        out_shape=jax.ShapeDtypeStruct((vocab, B), jnp.float32),
    )(tag_t, avg)


def kernel(bags, bag_emb, tag_emb):
    fake = _tc_table_rows(bag_emb.T)
    table_rows = fake.reshape(TR * 8, EMB)
    bags_flat = bags.reshape(B * CTX)
    ridx = (bags_flat & (TR - 1)) * 8 + (bags_flat >> 14)
    ridx_r = ridx.reshape(NW, N_CHUNKS, IDX_CHUNK)
    avg = _sc_avg(ridx_r, table_rows)
    return _tc_scores_t(tag_emb.T, avg).T


# speedup vs baseline: 1.1198x; 1.0118x over previous
import functools

import jax
import jax.numpy as jnp
from jax import lax
from jax.experimental import pallas as pl
from jax.experimental.pallas import tpu as pltpu
from jax.experimental.pallas import tpu_sc as plsc

B = 1024
CTX = 20
EMB = 16
VOCAB_FLAT = 100000 * EMB
NW = 32
BAGS_PER_W = B // NW
ROWS_PER_W = BAGS_PER_W * CTX
IDX_CHUNK = 128
N_CHUNKS = ROWS_PER_W // IDX_CHUNK


def _sc_avg_body(bags_hbm, table_hbm, out_hbm, idx_v, rows_v, avg_v, sem):
    wid = lax.axis_index("s") * 2 + lax.axis_index("c")
    pltpu.sync_copy(bags_hbm.at[wid], idx_v)
    for c in range(N_CHUNKS * IDX_CHUNK // 16):
        e = idx_v[c // 8, pl.ds((c % 8) * 16, 16)]
        idx_v[c // 8, pl.ds((c % 8) * 16, 16)] = (
            (e & (TR - 1)) * 8 + (e >> TR_LOG2)
        )
    copies = []
    for j in range(N_CHUNKS):
        copies.append(
            pltpu.async_copy(
                table_hbm.at[idx_v.at[j]],
                rows_v.at[pl.ds(j * IDX_CHUNK, IDX_CHUNK)],
                sem,
            )
        )
    for c in copies:
        c.wait()

    inv = jnp.full((EMB,), 1.0 / CTX, dtype=jnp.float32)

    def pool_one(i, _):
        acc = rows_v[i * CTX]
        for j in range(1, CTX):
            acc = acc + rows_v[i * CTX + j]
        avg_v[i] = acc * inv
        return _

    lax.fori_loop(0, BAGS_PER_W, pool_one, 0, unroll=4)
    pltpu.sync_copy(avg_v, out_hbm.at[pl.ds(wid * BAGS_PER_W, BAGS_PER_W)])


@jax.jit
def _sc_avg(bags_r, bag_emb):
    mesh = plsc.VectorSubcoreMesh(core_axis_name="c", subcore_axis_name="s")
    return pl.kernel(
        _sc_avg_body,
        out_type=jax.ShapeDtypeStruct((B, EMB), jnp.float32),
        mesh=mesh,
        scratch_types=[
            pltpu.VMEM((N_CHUNKS, IDX_CHUNK), jnp.int32),
            pltpu.VMEM((ROWS_PER_W, EMB), jnp.float32),
            pltpu.VMEM((BAGS_PER_W, EMB), jnp.float32),
            pltpu.SemaphoreType.DMA,
        ],
        compiler_params=pltpu.CompilerParams(use_tc_tiling_on_sc=False),
    )(bags_r, bag_emb)


TR = 16384
TR_LOG2 = 14
TPB = 4096
TNB = TR // TPB


def _relayout_body(*refs):
    ins, out_ref = refs[:-1], refs[-1]
    y = jnp.concatenate([r[...] for r in ins], axis=0)
    out_ref[...] = y.T


@jax.jit
def _tc_table_rows(table_t):
    nblk = pl.cdiv(table_t.shape[1], TPB) - 1
    specs = [
        pl.BlockSpec(
            (EMB, TPB), (lambda i, a=a: (0, jnp.minimum(a * TNB + i, nblk)))
        )
        for a in range(8)
    ]
    return pl.pallas_call(
        _relayout_body,
        grid=(TNB,),
        in_specs=specs,
        out_specs=pl.BlockSpec((TPB, 128), lambda i: (i, 0)),
        out_shape=jax.ShapeDtypeStruct((TR, 128), jnp.float32),
    )(*([table_t] * 8))


VB = 2048


def _matmul_body(tag_ref, avg_ref, out_ref):
    out_ref[...] = lax.dot_general(
        tag_ref[...],
        avg_ref[...],
        (((0,), (1,)), ((), ())),
        preferred_element_type=jnp.float32,
    )


@jax.jit
def _tc_scores_t(tag_t, avg):
    vocab = tag_t.shape[1]
    grid = (pl.cdiv(vocab, VB),)
    return pl.pallas_call(
        _matmul_body,
        grid=grid,
        in_specs=[
            pl.BlockSpec((EMB, VB), lambda i: (0, i)),
            pl.BlockSpec((B, EMB), lambda i: (0, 0)),
        ],
        out_specs=pl.BlockSpec((VB, B), lambda i: (i, 0)),
        out_shape=jax.ShapeDtypeStruct((vocab, B), jnp.float32),
    )(tag_t, avg)


def kernel(bags, bag_emb, tag_emb):
    fake = _tc_table_rows(bag_emb.T)
    table_rows = fake.reshape(TR * 8, EMB)
    bags_r = bags.reshape(NW, N_CHUNKS, IDX_CHUNK)
    avg = _sc_avg(bags_r, table_rows)
    return _tc_scores_t(tag_emb.T, avg).T


# speedup vs baseline: 1.1245x; 1.0042x over previous
import functools

import jax
import jax.numpy as jnp
from jax import lax
from jax.experimental import pallas as pl
from jax.experimental.pallas import tpu as pltpu
from jax.experimental.pallas import tpu_sc as plsc

B = 1024
CTX = 20
EMB = 16
VOCAB_FLAT = 100000 * EMB
NW = 32
BAGS_PER_W = B // NW
ROWS_PER_W = BAGS_PER_W * CTX
IDX_CHUNK = 128
N_CHUNKS = ROWS_PER_W // IDX_CHUNK


def _sc_avg_body(bags_hbm, table_hbm, out_hbm, raw_v, idx_v, rows_v, avg_v, sem):
    wid = lax.axis_index("s") * 2 + lax.axis_index("c")
    pltpu.sync_copy(bags_hbm.at[:, pl.ds(wid * BAGS_PER_W, BAGS_PER_W)], raw_v)
    for c in range(ROWS_PER_W // 16):
        e = raw_v[c // 2, pl.ds((c % 2) * 16, 16)]
        idx_v[c // 8, pl.ds((c % 8) * 16, 16)] = (
            (e & (TR - 1)) * 8 + (e >> TR_LOG2)
        )
    copies = []
    for j in range(N_CHUNKS):
        copies.append(
            pltpu.async_copy(
                table_hbm.at[idx_v.at[j]],
                rows_v.at[pl.ds(j * IDX_CHUNK, IDX_CHUNK)],
                sem,
            )
        )
    for c in copies:
        c.wait()

    inv = jnp.full((EMB,), 1.0 / CTX, dtype=jnp.float32)

    def pool_one(i, _):
        acc = rows_v[i]
        for j in range(1, CTX):
            acc = acc + rows_v[j * BAGS_PER_W + i]
        avg_v[i] = acc * inv
        return _

    lax.fori_loop(0, BAGS_PER_W, pool_one, 0, unroll=4)
    pltpu.sync_copy(avg_v, out_hbm.at[pl.ds(wid * BAGS_PER_W, BAGS_PER_W)])


@jax.jit
def _sc_avg(bags_r, bag_emb):
    mesh = plsc.VectorSubcoreMesh(core_axis_name="c", subcore_axis_name="s")
    return pl.kernel(
        _sc_avg_body,
        out_type=jax.ShapeDtypeStruct((B, EMB), jnp.float32),
        mesh=mesh,
        scratch_types=[
            pltpu.VMEM((CTX, BAGS_PER_W), jnp.int32),
            pltpu.VMEM((N_CHUNKS, IDX_CHUNK), jnp.int32),
            pltpu.VMEM((ROWS_PER_W, EMB), jnp.float32),
            pltpu.VMEM((BAGS_PER_W, EMB), jnp.float32),
            pltpu.SemaphoreType.DMA,
        ],
        compiler_params=pltpu.CompilerParams(use_tc_tiling_on_sc=False),
    )(bags_r, bag_emb)


TR = 16384
TR_LOG2 = 14
TPB = 4096
TNB = TR // TPB


def _relayout_body(*refs):
    ins, out_ref = refs[:-1], refs[-1]
    y = jnp.concatenate([r[...] for r in ins], axis=0)
    out_ref[...] = y.T


@jax.jit
def _tc_table_rows(table_t):
    nblk = pl.cdiv(table_t.shape[1], TPB) - 1
    specs = [
        pl.BlockSpec(
            (EMB, TPB), (lambda i, a=a: (0, jnp.minimum(a * TNB + i, nblk)))
        )
        for a in range(8)
    ]
    return pl.pallas_call(
        _relayout_body,
        grid=(TNB,),
        in_specs=specs,
        out_specs=pl.BlockSpec((TPB, 128), lambda i: (i, 0)),
        out_shape=jax.ShapeDtypeStruct((TR, 128), jnp.float32),
    )(*([table_t] * 8))


VB = 2048


def _matmul_body(tag_ref, avg_ref, out_ref):
    out_ref[...] = lax.dot_general(
        tag_ref[...],
        avg_ref[...],
        (((0,), (1,)), ((), ())),
        preferred_element_type=jnp.float32,
    )


@jax.jit
def _tc_scores_t(tag_t, avg):
    vocab = tag_t.shape[1]
    grid = (pl.cdiv(vocab, VB),)
    return pl.pallas_call(
        _matmul_body,
        grid=grid,
        in_specs=[
            pl.BlockSpec((EMB, VB), lambda i: (0, i)),
            pl.BlockSpec((B, EMB), lambda i: (0, 0)),
        ],
        out_specs=pl.BlockSpec((VB, B), lambda i: (i, 0)),
        out_shape=jax.ShapeDtypeStruct((vocab, B), jnp.float32),
    )(tag_t, avg)


def kernel(bags, bag_emb, tag_emb):
    fake = _tc_table_rows(bag_emb.T)
    table_rows = fake.reshape(TR * 8, EMB)
    avg = _sc_avg(bags.T, table_rows)
    return _tc_scores_t(tag_emb.T, avg).T
